# baseline (speedup 1.0000x reference)
import functools

import jax
import jax.numpy as jnp
from jax import lax
from jax.experimental import pallas as pl
from jax.experimental.pallas import tpu as pltpu
from jax.experimental.pallas import tpu_sc as plsc

N = 10000
E = 320000
G = 256
NA = 43
NC, NS, L = 2, 16, 16
NW = NC * NS
NPAD = 10240
NSL = NPAD // NS
BK = 128
NBLK = E // BK
BPW = NBLK // NW
CB = 8
ECH = CB * BK
NCF = BPW // CB
GB = G + L
XBN = 1024
EBL = 128

_mesh = plsc.VectorSubcoreMesh(core_axis_name="c", subcore_axis_name="s",
                               num_cores=NC, num_subcores=NS)
_params = pltpu.CompilerParams(use_tc_tiling_on_sc=False,
                               needs_layout_passes=False)


def _ii():
    return lax.iota(jnp.int32, L)


def _xk_body(x_ref, at_ref, pk_ref):
    v = x_ref[...]
    lid = lax.broadcasted_iota(jnp.int32, v.shape, 1)

    def amax(lo, hi):
        vm = jnp.where((lid >= lo) & (lid < hi), v, -jnp.inf)
        m = jnp.max(vm, axis=1, keepdims=True)
        return jnp.min(jnp.where(vm == m, lid, 128), axis=1)

    at_ref[...] = amax(0, NA).reshape(XBN // 128, 128)
    pk = (amax(43, 50) - 43) + ((amax(126, 128) - 126) << 8)
    pk_ref[...] = pk.reshape(XBN // 128, 128)


_xk = pl.pallas_call(
    _xk_body,
    grid=(NPAD // XBN,),
    in_specs=[pl.BlockSpec((XBN, 128), lambda i: (i, 0))],
    out_specs=(pl.BlockSpec((XBN // 128, 128), lambda i: (i, 0)),
               pl.BlockSpec((XBN // 128, 128), lambda i: (i, 0))),
    out_shape=(jax.ShapeDtypeStruct((NPAD // 128, 128), jnp.int32),
               jax.ShapeDtypeStruct((NPAD // 128, 128), jnp.int32)),
)


def _btk_body(ef_ref, bt_ref):
    m = ef_ref[0, :]
    bt = jnp.zeros(m.shape, jnp.int32)
    for j in range(1, 5):
        v = ef_ref[j, :]
        gt = v > m
        m = jnp.where(gt, v, m)
        bt = jnp.where(gt, j, bt)
    bt_ref[...] = bt.reshape(EBL, 128)


_btk = pl.pallas_call(
    _btk_body,
    grid=(NBLK // EBL + 1,),
    in_specs=[pl.BlockSpec((8, EBL * 128), lambda i: (0, i))],
    out_specs=pl.BlockSpec((EBL, 128), lambda i: (i, 0)),
    out_shape=jax.ShapeDtypeStruct((NBLK, 128), jnp.int32),
)


@functools.partial(
    pl.kernel,
    out_type=jax.ShapeDtypeStruct((NC * 4 * NPAD,), jnp.float32),
    mesh=_mesh,
    compiler_params=_params,
    scratch_types=[
        pltpu.VMEM((NPAD,), jnp.int32),
        pltpu.VMEM((4 * NPAD,), jnp.float32),
        pltpu.VMEM((2 * 2 * ECH,), jnp.int32),
        pltpu.VMEM((2 * ECH,), jnp.int32),
        pltpu.VMEM((2 * ECH,), jnp.float32),
        pltpu.VMEM((2 * ECH,), jnp.int32),
        pltpu.VMEM((NS, NSL), jnp.float32),
        pltpu.VMEM_SHARED((NS, 4 * NPAD), jnp.float32),
        pltpu.SemaphoreType.DMA,
    ],
)
def _agg(at_hbm, ei_hbm, isg_hbm, elp_hbm, bt_hbm,
         partials_hbm,
         atoms_v, acc, sdb, isgb, elpb, btb, redb,
         sh_acc, sem):
    c = lax.axis_index("c")
    s = lax.axis_index("s")
    wid = s * NC + c
    ii = _ii()

    at_cp = pltpu.async_copy(at_hbm, atoms_v, sem)
    z = jnp.zeros((L,), jnp.float32)

    def zero_body(i, carry):
        for u in range(16):
            acc[pl.ds((i * 16 + u) * L, L)] = z
        return carry

    lax.fori_loop(0, 4 * NPAD // (16 * L), zero_body, 0)
    at_cp.wait()

    wblk0 = wid * BPW + jnp.minimum(wid, 4)
    wcnt = BPW + jnp.where(wid < 4, 1, 0)
    tail = wcnt - NCF * CB
    ones_f = jnp.full((L,), 1.0, jnp.float32)

    def issue(ch, sl):
        e0 = (wblk0 + ch * CB) * BK
        pltpu.async_copy(ei_hbm.at[pl.ds(e0 * 2, ECH * 2)],
                         sdb.at[pl.ds(sl * 2 * ECH, ECH * 2)], sem)
        o = sl * ECH
        pltpu.async_copy(isg_hbm.at[pl.ds(e0, ECH)],
                         isgb.at[pl.ds(o, ECH)], sem)
        pltpu.async_copy(elp_hbm.at[pl.ds(e0, ECH)],
                         elpb.at[pl.ds(o, ECH)], sem)
        pltpu.async_copy(bt_hbm.at[pl.ds(e0, ECH)],
                         btb.at[pl.ds(o, ECH)], sem)

    def drain(sl):
        o = sl * ECH
        pltpu.make_async_copy(ei_hbm.at[pl.ds(0, ECH * 2)],
                              sdb.at[pl.ds(sl * 2 * ECH, ECH * 2)],
                              sem).wait()
        pltpu.make_async_copy(isg_hbm.at[pl.ds(0, ECH)],
                              isgb.at[pl.ds(o, ECH)], sem).wait()
        pltpu.make_async_copy(elp_hbm.at[pl.ds(0, ECH)],
                              elpb.at[pl.ds(o, ECH)], sem).wait()
        pltpu.make_async_copy(bt_hbm.at[pl.ds(0, ECH)],
                              btb.at[pl.ds(o, ECH)], sem).wait()

    def egroups(sd_base, base, ngroups):
        def egroup(g, carry2):
            o = base + g * L
            sd = sd_base + (g >> 3) * (2 * BK) + (g & 7) * L
            src16 = sdb[pl.ds(sd, L)]
            dst16 = sdb[pl.ds(sd + BK, L)]
            isg16 = isgb[pl.ds(o, L)]
            elp16 = elpb[pl.ds(o, L)]
            bt = btb[pl.ds(o, L)]
            sa = plsc.load_gather(atoms_v, [src16])
            da = plsc.load_gather(atoms_v, [dst16])
            isar = jnp.where((sa != 42) & (bt == 4) & (da != 42), 1.0, 0.0)
            bt2 = jnp.where(bt == 4, 1, bt)
            bt2 = jnp.where(sa == 42, 0, bt2)
            bt2 = jnp.where(isg16 == -1, 0, bt2).astype(jnp.float32)
            plsc.addupdate_scatter(acc, [dst16], isar)
            plsc.addupdate_scatter(acc, [dst16 + NPAD], bt2)
            plsc.addupdate_scatter(acc, [dst16 + 2 * NPAD], ones_f)
            plsc.addupdate_scatter(acc, [dst16 + 3 * NPAD], elp16)
            return carry2

        lax.fori_loop(0, ngroups, egroup, 0)

    with jax.named_scope("p1_edges"):
        issue(0, 0)

        def chunk(ch, carry):
            sl = ch & 1
            drain(sl)
            issue(jnp.minimum(ch + 1, NCF - 1), 1 - sl)
            egroups(sl * 2 * ECH, sl * ECH, ECH // L)
            return carry

        lax.fori_loop(0, NCF, chunk, 0)
        drain(NCF & 1)

        e0t = (wblk0 + NCF * CB) * BK

        def tail_body(b, carry):
            e0 = e0t + b * BK
            pltpu.sync_copy(ei_hbm.at[pl.ds(e0 * 2, BK * 2)],
                            sdb.at[pl.ds(0, BK * 2)])
            pltpu.sync_copy(isg_hbm.at[pl.ds(e0, BK)], isgb.at[pl.ds(0, BK)])
            pltpu.sync_copy(elp_hbm.at[pl.ds(e0, BK)], elpb.at[pl.ds(0, BK)])
            pltpu.sync_copy(bt_hbm.at[pl.ds(e0, BK)], btb.at[pl.ds(0, BK)])
            egroups(0, 0, BK // L)
            return carry

        lax.fori_loop(0, tail, tail_body, 0)

    with jax.named_scope("p2_reduce"):
        pltpu.sync_copy(acc, sh_acc.at[s])
        plsc.subcore_barrier()
        for k in range(4):
            pltpu.sync_copy(
                sh_acc.at[:, pl.ds(k * NPAD + s * NSL, NSL)], redb)

            def red_body(g, carry):
                tot = redb[0, pl.ds(g * L, L)]
                for t in range(1, NS):
                    tot = tot + redb[t, pl.ds(g * L, L)]
                acc[pl.ds(k * NPAD + s * NSL + g * L, L)] = tot
                return carry

            lax.fori_loop(0, NSL // L, red_body, 0)
        for k in range(4):
            pltpu.sync_copy(
                acc.at[pl.ds(k * NPAD + s * NSL, NSL)],
                partials_hbm.at[pl.ds(((s * NC + c) * 4 + k) * NSL, NSL)])


@functools.partial(
    pl.kernel,
    out_type=jax.ShapeDtypeStruct((L,), jnp.float32),
    mesh=_mesh,
    compiler_params=_params,
    scratch_types=[
        pltpu.VMEM((NSL,), jnp.int32),
        pltpu.VMEM((NSL,), jnp.float32),
        pltpu.VMEM((NSL,), jnp.int32),
        pltpu.VMEM((NSL,), jnp.int32),
        pltpu.VMEM((8 * NSL,), jnp.float32),
        pltpu.VMEM((128,), jnp.float32),
        pltpu.VMEM((GB,), jnp.float32),
        pltpu.VMEM((NS, GB), jnp.float32),
        pltpu.VMEM((2 * GB,), jnp.float32),
        pltpu.VMEM((NS, 2 * GB), jnp.float32),
        pltpu.VMEM((L,), jnp.float32),
        pltpu.VMEM_SHARED((NS, GB), jnp.float32),
        pltpu.VMEM_SHARED((NS, 2 * GB), jnp.float32),
        pltpu.SemaphoreType.DMA,
    ],
)
def _fin(at_hbm, pk_hbm, part_hbm, nlp_hbm, gid_hbm, tbl_hbm,
         out_hbm,
         gidb, nlpb, atb, pkb, pab, tablev, binc, bfull, pools, pfb,
         outv, sh_binc, sh_pools, sem):
    c = lax.axis_index("c")
    s = lax.axis_index("s")
    ii = _ii()
    n0 = s * NSL
    cps = [
        pltpu.async_copy(gid_hbm.at[pl.ds(n0, NSL)], gidb, sem),
        pltpu.async_copy(nlp_hbm.at[pl.ds(n0, NSL)], nlpb, sem),
        pltpu.async_copy(at_hbm.at[pl.ds(n0, NSL)], atb, sem),
        pltpu.async_copy(pk_hbm.at[pl.ds(n0, NSL)], pkb, sem),
        pltpu.async_copy(part_hbm.at[pl.ds(s * 8 * NSL, 8 * NSL)], pab, sem),
        pltpu.async_copy(tbl_hbm, tablev, sem),
    ]
    for cp in cps:
        cp.wait()

    z = jnp.zeros((L,), jnp.float32)
    for i in range(GB // L):
        binc[pl.ds(i * L, L)] = z
    ones_f = jnp.full((L,), 1.0, jnp.float32)

    def bc_body(g, carry):
        gid16 = gidb[pl.ds(g * L, L)]
        plsc.addupdate_scatter(binc, [gid16], ones_f)
        return carry

    lax.fori_loop(0, NSL // L, bc_body, 0)
    pltpu.sync_copy(binc, sh_binc.at[s])
    plsc.subcore_barrier()
    pltpu.sync_copy(sh_binc, bfull)
    for i in range(GB // L):
        tot = bfull[0, pl.ds(i * L, L)]
        for t in range(1, NS):
            tot = tot + bfull[t, pl.ds(i * L, L)]
        binc[pl.ds(i * L, L)] = tot

    for i in range(2 * GB // L):
        pools[pl.ds(i * L, L)] = z

    def node_body(g, carry):
        sl = pl.ds(g * L, L)
        gid16 = gidb[sl]
        nlp16 = nlpb[sl]
        at16 = atb[sl]
        pk16 = pkb[sl]
        ch16 = (pk16 & 255) - 3
        ar16 = pk16 >> 8

        def agg(k):
            return (pab[pl.ds(k * NSL + g * L, L)] +
                    pab[pl.ds((4 + k) * NSL + g * L, L)])

        har = agg(0)
        nb = agg(1)
        deg = agg(2)
        selp = agg(3)
        af = jnp.where((har > 0.0) != (ar16 > 0), 1.0, 0.0)
        at16c = jnp.minimum(jnp.maximum(at16, 0), NA - 1)
        mb16 = plsc.load_gather(tablev, [at16c])
        vf = jnp.where((nb - ch16.astype(jnp.float32)) > mb16, 1.0, 0.0)
        bf = jnp.where((at16 == 42) != (nb == 0.0), 1.0, 0.0)
        reward = -af - 2.0 * vf - 3.0 * bf
        nn16 = plsc.load_gather(binc, [gid16])
        reward = jnp.where((nn16 == 1.0) & (at16 == 42), -4.0, reward)
        mselp = selp / jnp.maximum(deg, 1.0)
        loss = -(nlp16 + mselp) * reward
        plsc.addupdate_scatter(pools, [gid16], loss)
        plsc.addupdate_scatter(pools, [gid16 + GB], reward)
        return carry

    lax.fori_loop(0, NSL // L, node_body, 0)
    pltpu.sync_copy(pools, sh_pools.at[s])
    plsc.subcore_barrier()

    pltpu.sync_copy(sh_pools, pfb)
    suml = jnp.zeros((L,), jnp.float32)
    sumr = jnp.zeros((L,), jnp.float32)
    for i in range(G // L):
        tl = pfb[0, pl.ds(i * L, L)]
        tr = pfb[0, pl.ds(GB + i * L, L)]
        for t in range(1, NS):
            tl = tl + pfb[t, pl.ds(i * L, L)]
            tr = tr + pfb[t, pl.ds(GB + i * L, L)]
        cnt = jnp.maximum(binc[pl.ds(i * L, L)], 1.0)
        suml = suml + tl / cnt
        sumr = sumr + tr / cnt
    tl_s = jnp.sum(suml) * (1.0 / G)
    tr_s = jnp.sum(sumr) * (1.0 / G)
    outv[...] = jnp.where(ii == 0, tl_s, jnp.where(ii == 1, tr_s, 0.0))

    @pl.when((s == 0) & (c == 0))
    def _():
        pltpu.sync_copy(outv, out_hbm)


def kernel(x, edge_feats, node_logprobs, edge_logprobs, max_bonds_table,
           edge_index, isgen, graph_ids):
    ei1 = edge_index.reshape(2, NBLK, BK).transpose(1, 0, 2).reshape(-1)
    gid_p = jnp.concatenate(
        [graph_ids, jnp.full((NPAD - N,), G, jnp.int32)])
    nlp_p = jnp.concatenate(
        [node_logprobs, jnp.zeros((NPAD - N,), jnp.float32)])
    tbl_p = jnp.concatenate(
        [max_bonds_table, jnp.zeros((128 - NA,), jnp.float32)])
    at2d, pk2d = _xk(x)
    bt2d = _btk(edge_feats.T)
    at1 = at2d.reshape(NPAD)
    pk1 = pk2d.reshape(NPAD)
    bt1 = bt2d.reshape(E)
    partials = _agg(at1, ei1, isgen, edge_logprobs, bt1)
    out = _fin(at1, pk1, partials, nlp_p, gid_p, tbl_p)
    return (out[0], out[1])


# speedup vs baseline: 1.0482x; 1.0482x over previous
import functools

import jax
import jax.numpy as jnp
from jax import lax
from jax.experimental import pallas as pl
from jax.experimental.pallas import tpu as pltpu
from jax.experimental.pallas import tpu_sc as plsc

N = 10000
E = 320000
G = 256
NA = 43
NC, NS, L = 2, 16, 16
NW = NC * NS
NPAD = 10240
NSL = NPAD // NS
BK = 128
NBLK = E // BK
BPW = NBLK // NW
CB = 4
ECH = CB * BK
NCF = BPW // CB
GB = G + L
XBN = 1024
EBL = 128

_mesh = plsc.VectorSubcoreMesh(core_axis_name="c", subcore_axis_name="s",
                               num_cores=NC, num_subcores=NS)
_params = pltpu.CompilerParams(use_tc_tiling_on_sc=False,
                               needs_layout_passes=False)


def _ii():
    return lax.iota(jnp.int32, L)


def _xk_body(x_ref, at_ref, pk_ref):
    v = x_ref[...]
    lid = lax.broadcasted_iota(jnp.int32, v.shape, 1)

    def amax(lo, hi):
        vm = jnp.where((lid >= lo) & (lid < hi), v, -jnp.inf)
        m = jnp.max(vm, axis=1, keepdims=True)
        return jnp.min(jnp.where(vm == m, lid, 128), axis=1)

    at_ref[...] = amax(0, NA).reshape(XBN // 128, 128)
    pk = (amax(43, 50) - 43) + ((amax(126, 128) - 126) << 8)
    pk_ref[...] = pk.reshape(XBN // 128, 128)


_xk = pl.pallas_call(
    _xk_body,
    grid=(NPAD // XBN,),
    in_specs=[pl.BlockSpec((XBN, 128), lambda i: (i, 0))],
    out_specs=(pl.BlockSpec((XBN // 128, 128), lambda i: (i, 0)),
               pl.BlockSpec((XBN // 128, 128), lambda i: (i, 0))),
    out_shape=(jax.ShapeDtypeStruct((NPAD // 128, 128), jnp.int32),
               jax.ShapeDtypeStruct((NPAD // 128, 128), jnp.int32)),
)


def _btk_body(ef_ref, bt_ref):
    m = ef_ref[0, :]
    bt = jnp.zeros(m.shape, jnp.int32)
    for j in range(1, 5):
        v = ef_ref[j, :]
        gt = v > m
        m = jnp.where(gt, v, m)
        bt = jnp.where(gt, j, bt)
    bt_ref[...] = bt.reshape(EBL, 128)


_btk = pl.pallas_call(
    _btk_body,
    grid=(NBLK // EBL + 1,),
    in_specs=[pl.BlockSpec((8, EBL * 128), lambda i: (0, i))],
    out_specs=pl.BlockSpec((EBL, 128), lambda i: (i, 0)),
    out_shape=jax.ShapeDtypeStruct((NBLK, 128), jnp.int32),
)


@functools.partial(
    pl.kernel,
    out_type=jax.ShapeDtypeStruct((NC * 4 * NPAD,), jnp.float32),
    mesh=_mesh,
    compiler_params=_params,
    scratch_types=[
        pltpu.VMEM((NPAD,), jnp.int32),
        pltpu.VMEM((4 * NPAD,), jnp.float32),
        pltpu.VMEM((2 * 2 * ECH,), jnp.int32),
        pltpu.VMEM((2 * ECH,), jnp.int32),
        pltpu.VMEM((2 * ECH,), jnp.float32),
        pltpu.VMEM((2 * ECH,), jnp.int32),
        pltpu.VMEM((NS, NSL), jnp.float32),
        pltpu.VMEM_SHARED((NS, 4 * NPAD), jnp.float32),
        pltpu.SemaphoreType.DMA,
    ],
)
def _agg(at_hbm, ei_hbm, isg_hbm, elp_hbm, bt_hbm,
         partials_hbm,
         atoms_v, acc, sdb, isgb, elpb, btb, redb,
         sh_acc, sem):
    c = lax.axis_index("c")
    s = lax.axis_index("s")
    wid = s * NC + c
    ii = _ii()

    at_cp = pltpu.async_copy(at_hbm, atoms_v, sem)
    z = jnp.zeros((L,), jnp.float32)

    def zero_body(i, carry):
        for u in range(16):
            acc[pl.ds((i * 16 + u) * L, L)] = z
        return carry

    lax.fori_loop(0, 4 * NPAD // (16 * L), zero_body, 0)
    at_cp.wait()

    wblk0 = wid * BPW + jnp.minimum(wid, 4)
    wcnt = BPW + jnp.where(wid < 4, 1, 0)
    tail = wcnt - NCF * CB
    ones_f = jnp.full((L,), 1.0, jnp.float32)

    def issue(ch, sl):
        e0 = (wblk0 + ch * CB) * BK
        pltpu.async_copy(ei_hbm.at[pl.ds(e0 * 2, ECH * 2)],
                         sdb.at[pl.ds(sl * 2 * ECH, ECH * 2)], sem)
        o = sl * ECH
        pltpu.async_copy(isg_hbm.at[pl.ds(e0, ECH)],
                         isgb.at[pl.ds(o, ECH)], sem)
        pltpu.async_copy(elp_hbm.at[pl.ds(e0, ECH)],
                         elpb.at[pl.ds(o, ECH)], sem)
        pltpu.async_copy(bt_hbm.at[pl.ds(e0, ECH)],
                         btb.at[pl.ds(o, ECH)], sem)

    def drain(sl):
        o = sl * ECH
        pltpu.make_async_copy(ei_hbm.at[pl.ds(0, ECH * 2)],
                              sdb.at[pl.ds(sl * 2 * ECH, ECH * 2)],
                              sem).wait()
        pltpu.make_async_copy(isg_hbm.at[pl.ds(0, ECH)],
                              isgb.at[pl.ds(o, ECH)], sem).wait()
        pltpu.make_async_copy(elp_hbm.at[pl.ds(0, ECH)],
                              elpb.at[pl.ds(o, ECH)], sem).wait()
        pltpu.make_async_copy(bt_hbm.at[pl.ds(0, ECH)],
                              btb.at[pl.ds(o, ECH)], sem).wait()

    def egroups(sd_base, base, ngroups):
        def egroup(g, carry2):
            o = base + g * L
            sd = sd_base + (g >> 3) * (2 * BK) + (g & 7) * L
            src16 = sdb[pl.ds(sd, L)]
            dst16 = sdb[pl.ds(sd + BK, L)]
            isg16 = isgb[pl.ds(o, L)]
            elp16 = elpb[pl.ds(o, L)]
            bt = btb[pl.ds(o, L)]
            sa = plsc.load_gather(atoms_v, [src16])
            da = plsc.load_gather(atoms_v, [dst16])
            isar = jnp.where((sa != 42) & (bt == 4) & (da != 42), 1.0, 0.0)
            bt2 = jnp.where(bt == 4, 1, bt)
            bt2 = jnp.where(sa == 42, 0, bt2)
            bt2 = jnp.where(isg16 == -1, 0, bt2).astype(jnp.float32)
            plsc.addupdate_scatter(acc, [dst16], isar)
            plsc.addupdate_scatter(acc, [dst16 + NPAD], bt2)
            plsc.addupdate_scatter(acc, [dst16 + 2 * NPAD], ones_f)
            plsc.addupdate_scatter(acc, [dst16 + 3 * NPAD], elp16)
            return carry2

        lax.fori_loop(0, ngroups, egroup, 0)

    with jax.named_scope("p1_edges"):
        issue(0, 0)

        def chunk(ch, carry):
            sl = ch & 1
            drain(sl)
            issue(jnp.minimum(ch + 1, NCF - 1), 1 - sl)
            egroups(sl * 2 * ECH, sl * ECH, ECH // L)
            return carry

        lax.fori_loop(0, NCF, chunk, 0)
        drain(NCF & 1)

        e0t = (wblk0 + NCF * CB) * BK

        def tail_body(b, carry):
            e0 = e0t + b * BK
            pltpu.sync_copy(ei_hbm.at[pl.ds(e0 * 2, BK * 2)],
                            sdb.at[pl.ds(0, BK * 2)])
            pltpu.sync_copy(isg_hbm.at[pl.ds(e0, BK)], isgb.at[pl.ds(0, BK)])
            pltpu.sync_copy(elp_hbm.at[pl.ds(e0, BK)], elpb.at[pl.ds(0, BK)])
            pltpu.sync_copy(bt_hbm.at[pl.ds(e0, BK)], btb.at[pl.ds(0, BK)])
            egroups(0, 0, BK // L)
            return carry

        lax.fori_loop(0, tail, tail_body, 0)

    with jax.named_scope("p2_reduce"):
        pltpu.sync_copy(acc, sh_acc.at[s])
        plsc.subcore_barrier()
        for k in range(4):
            pltpu.sync_copy(
                sh_acc.at[:, pl.ds(k * NPAD + s * NSL, NSL)], redb)

            def red_body(g, carry):
                tot = redb[0, pl.ds(g * L, L)]
                for t in range(1, NS):
                    tot = tot + redb[t, pl.ds(g * L, L)]
                acc[pl.ds(k * NPAD + s * NSL + g * L, L)] = tot
                return carry

            lax.fori_loop(0, NSL // L, red_body, 0)
        for k in range(4):
            pltpu.sync_copy(
                acc.at[pl.ds(k * NPAD + s * NSL, NSL)],
                partials_hbm.at[pl.ds(((s * NC + c) * 4 + k) * NSL, NSL)])


@functools.partial(
    pl.kernel,
    out_type=jax.ShapeDtypeStruct((L,), jnp.float32),
    mesh=_mesh,
    compiler_params=_params,
    scratch_types=[
        pltpu.VMEM((NSL,), jnp.int32),
        pltpu.VMEM((NSL,), jnp.float32),
        pltpu.VMEM((NSL,), jnp.int32),
        pltpu.VMEM((NSL,), jnp.int32),
        pltpu.VMEM((8 * NSL,), jnp.float32),
        pltpu.VMEM((128,), jnp.float32),
        pltpu.VMEM((GB,), jnp.float32),
        pltpu.VMEM((NS, GB), jnp.float32),
        pltpu.VMEM((2 * GB,), jnp.float32),
        pltpu.VMEM((NS, 2 * GB), jnp.float32),
        pltpu.VMEM((L,), jnp.float32),
        pltpu.VMEM_SHARED((NS, GB), jnp.float32),
        pltpu.VMEM_SHARED((NS, 2 * GB), jnp.float32),
        pltpu.SemaphoreType.DMA,
    ],
)
def _fin(at_hbm, pk_hbm, part_hbm, nlp_hbm, gid_hbm, tbl_hbm,
         out_hbm,
         gidb, nlpb, atb, pkb, pab, tablev, binc, bfull, pools, pfb,
         outv, sh_binc, sh_pools, sem):
    c = lax.axis_index("c")
    s = lax.axis_index("s")
    ii = _ii()
    n0 = s * NSL
    cps = [
        pltpu.async_copy(gid_hbm.at[pl.ds(n0, NSL)], gidb, sem),
        pltpu.async_copy(nlp_hbm.at[pl.ds(n0, NSL)], nlpb, sem),
        pltpu.async_copy(at_hbm.at[pl.ds(n0, NSL)], atb, sem),
        pltpu.async_copy(pk_hbm.at[pl.ds(n0, NSL)], pkb, sem),
        pltpu.async_copy(part_hbm.at[pl.ds(s * 8 * NSL, 8 * NSL)], pab, sem),
        pltpu.async_copy(tbl_hbm, tablev, sem),
    ]
    for cp in cps:
        cp.wait()

    z = jnp.zeros((L,), jnp.float32)
    for i in range(GB // L):
        binc[pl.ds(i * L, L)] = z
    ones_f = jnp.full((L,), 1.0, jnp.float32)

    def bc_body(g, carry):
        gid16 = gidb[pl.ds(g * L, L)]
        plsc.addupdate_scatter(binc, [gid16], ones_f)
        return carry

    lax.fori_loop(0, NSL // L, bc_body, 0)
    pltpu.sync_copy(binc, sh_binc.at[s])
    plsc.subcore_barrier()
    pltpu.sync_copy(sh_binc, bfull)
    for i in range(GB // L):
        tot = bfull[0, pl.ds(i * L, L)]
        for t in range(1, NS):
            tot = tot + bfull[t, pl.ds(i * L, L)]
        binc[pl.ds(i * L, L)] = tot

    for i in range(2 * GB // L):
        pools[pl.ds(i * L, L)] = z

    def node_body(g, carry):
        sl = pl.ds(g * L, L)
        gid16 = gidb[sl]
        nlp16 = nlpb[sl]
        at16 = atb[sl]
        pk16 = pkb[sl]
        ch16 = (pk16 & 255) - 3
        ar16 = pk16 >> 8

        def agg(k):
            return (pab[pl.ds(k * NSL + g * L, L)] +
                    pab[pl.ds((4 + k) * NSL + g * L, L)])

        har = agg(0)
        nb = agg(1)
        deg = agg(2)
        selp = agg(3)
        af = jnp.where((har > 0.0) != (ar16 > 0), 1.0, 0.0)
        at16c = jnp.minimum(jnp.maximum(at16, 0), NA - 1)
        mb16 = plsc.load_gather(tablev, [at16c])
        vf = jnp.where((nb - ch16.astype(jnp.float32)) > mb16, 1.0, 0.0)
        bf = jnp.where((at16 == 42) != (nb == 0.0), 1.0, 0.0)
        reward = -af - 2.0 * vf - 3.0 * bf
        nn16 = plsc.load_gather(binc, [gid16])
        reward = jnp.where((nn16 == 1.0) & (at16 == 42), -4.0, reward)
        mselp = selp / jnp.maximum(deg, 1.0)
        loss = -(nlp16 + mselp) * reward
        plsc.addupdate_scatter(pools, [gid16], loss)
        plsc.addupdate_scatter(pools, [gid16 + GB], reward)
        return carry

    lax.fori_loop(0, NSL // L, node_body, 0)
    pltpu.sync_copy(pools, sh_pools.at[s])
    plsc.subcore_barrier()

    pltpu.sync_copy(sh_pools, pfb)
    suml = jnp.zeros((L,), jnp.float32)
    sumr = jnp.zeros((L,), jnp.float32)
    for i in range(G // L):
        tl = pfb[0, pl.ds(i * L, L)]
        tr = pfb[0, pl.ds(GB + i * L, L)]
        for t in range(1, NS):
            tl = tl + pfb[t, pl.ds(i * L, L)]
            tr = tr + pfb[t, pl.ds(GB + i * L, L)]
        cnt = jnp.maximum(binc[pl.ds(i * L, L)], 1.0)
        suml = suml + tl / cnt
        sumr = sumr + tr / cnt
    tl_s = jnp.sum(suml) * (1.0 / G)
    tr_s = jnp.sum(sumr) * (1.0 / G)
    outv[...] = jnp.where(ii == 0, tl_s, jnp.where(ii == 1, tr_s, 0.0))

    @pl.when((s == 0) & (c == 0))
    def _():
        pltpu.sync_copy(outv, out_hbm)


def kernel(x, edge_feats, node_logprobs, edge_logprobs, max_bonds_table,
           edge_index, isgen, graph_ids):
    ei1 = edge_index.reshape(2, NBLK, BK).transpose(1, 0, 2).reshape(-1)
    gid_p = jnp.concatenate(
        [graph_ids, jnp.full((NPAD - N,), G, jnp.int32)])
    nlp_p = jnp.concatenate(
        [node_logprobs, jnp.zeros((NPAD - N,), jnp.float32)])
    tbl_p = jnp.concatenate(
        [max_bonds_table, jnp.zeros((128 - NA,), jnp.float32)])
    at2d, pk2d = _xk(x)
    bt2d = _btk(edge_feats.T)
    at1 = at2d.reshape(NPAD)
    pk1 = pk2d.reshape(NPAD)
    bt1 = bt2d.reshape(E)
    partials = _agg(at1, ei1, isgen, edge_logprobs, bt1)
    out = _fin(at1, pk1, partials, nlp_p, gid_p, tbl_p)
    return (out[0], out[1])
